# baseline (speedup 1.0000x reference)
import jax
import jax.numpy as jnp
from jax import lax
from jax.experimental import pallas as pl
from jax.experimental.pallas import tpu as pltpu
from jax.experimental.pallas import tpu_sc as plsc

_B = 16384
_V1 = 100001
_D = 64
_NN = 22
_NCAT = 6
_CV1 = 101
_CD = 8
_DOUT = _D + _NN + _NCAT * _CD

_NCORES = 2
_NSUB = 16
_NW = _NCORES * _NSUB
_RB = _B // _NW
_Q = 64
_NQ = _RB // _Q


def _extract_quarter(paired_v, out_blk, q):
    @plsc.parallel_loop(0, (_Q // 16) * _D, unroll=8)
    def _(t):
        g = t // _D
        d0 = lax.rem(t, _D)
        b0 = q * _Q + g * 16
        lanes_i = lax.iota(jnp.int32, 16)
        dvec = (lanes_i + d0) & (_D - 1)
        rows = lanes_i + g * 16
        vals = plsc.load_gather(paired_v, [rows, dvec])
        plsc.store_scatter(out_blk, [dvec, lanes_i + b0], vals)


def _body(fid_hbm, numt_hbm, catt_hbm, ftab_hbm, ctab_hbm, smalls_hbm,
          out_hbm,
          fid_v, paired_a, paired_b, out_blk, numt_v,
          catid_v, ctab_v, smalls_v, gsem):
    wid = lax.axis_index("s") * _NCORES + lax.axis_index("c")
    base = wid * _RB

    pltpu.sync_copy(fid_hbm.at[pl.ds(base, _RB)], fid_v)

    def fire(q, buf):
        return pltpu.async_copy(
            ftab_hbm.at[fid_v.at[pl.ds(q * _Q, _Q)]], buf, gsem)

    d0 = fire(0, paired_a)

    pltpu.sync_copy(catt_hbm.at[:, pl.ds(base, _RB)], catid_v)
    pltpu.sync_copy(numt_hbm.at[:, pl.ds(base, _RB)], numt_v)
    pltpu.sync_copy(ctab_hbm, ctab_v)
    pltpu.sync_copy(smalls_hbm, smalls_v)

    @plsc.parallel_loop(0, _RB // 16, unroll=2)
    def _(g):
        for f in range(_NCAT):
            ids = catid_v[f, pl.ds(g * 16, 16)]
            addr = ids * (_CD + 1) + (f * _CV1 * (_CD + 1))
            for d in range(_CD):
                vals = plsc.load_gather(ctab_v, [addr + d])
                out_blk[_D + _NN + f * _CD + d, pl.ds(g * 16, 16)] = vals

    @plsc.parallel_loop(0, _RB // 16, unroll=2)
    def _(g):
        sv0 = smalls_v[pl.ds(_D, 16)]
        sv1 = smalls_v[pl.ds(_D + 16, 16)]
        bv0 = smalls_v[pl.ds(_D + 32, 16)]
        bv1 = smalls_v[pl.ds(_D + 48, 16)]
        for f in range(_NN):
            x = numt_v[f, pl.ds(g * 16, 16)]
            s = sv0[f] if f < 16 else sv1[f - 16]
            b = bv0[f] if f < 16 else bv1[f - 16]
            out_blk[_D + f, pl.ds(g * 16, 16)] = x * s - b

    bufs = (paired_a, paired_b)
    descs = [d0] + [None] * (_NQ - 1)
    for c in range(_NQ):
        descs[c].wait()
        if c + 1 < _NQ:
            descs[c + 1] = fire(c + 1, bufs[(c + 1) % 2])
        _extract_quarter(bufs[c % 2], out_blk, c)

    pltpu.sync_copy(out_blk, out_hbm.at[:, pl.ds(base, _RB)])


_sc_call = pl.kernel(
    _body,
    out_type=jax.ShapeDtypeStruct((_DOUT, _B), jnp.float32),
    mesh=plsc.VectorSubcoreMesh(
        core_axis_name="c", subcore_axis_name="s",
        num_cores=_NCORES, num_subcores=_NSUB),
    compiler_params=pltpu.CompilerParams(needs_layout_passes=False),
    scratch_types=[
        pltpu.VMEM((_RB,), jnp.int32),
        pltpu.VMEM((_Q, 2 * _D), jnp.float32),
        pltpu.VMEM((_Q, 2 * _D), jnp.float32),
        pltpu.VMEM((_DOUT, _RB), jnp.float32),
        pltpu.VMEM((_NN, _RB), jnp.float32),
        pltpu.VMEM((_NCAT, _RB), jnp.int32),
        pltpu.VMEM((_NCAT * _CV1 * (_CD + 1),), jnp.float32),
        pltpu.VMEM((2 * _D,), jnp.float32),
        pltpu.SemaphoreType.DMA,
    ],
)


@jax.jit
def kernel(food_id, num_feats, cat_ids, food_table, cat_tables, norm_mean,
           norm_std):
    fid = food_id.astype(jnp.int32)
    ftab2 = jnp.pad(food_table, ((0, 0), (0, _D)))
    numt = num_feats.T
    catt = cat_ids.astype(jnp.int32).T
    ctab = jnp.pad(cat_tables, ((0, 0), (0, 0), (0, 1))).reshape(
        _NCAT * _CV1 * (_CD + 1))
    smalls = jnp.concatenate([
        food_table[_V1 - 1],
        jnp.pad((1.0 / norm_std).astype(jnp.float32), (0, 32 - _NN)),
        jnp.pad((norm_mean / norm_std).astype(jnp.float32),
                (0, 32 - _NN)),
    ])
    out_t = _sc_call(fid, numt, catt, ftab2, ctab, smalls)
    return out_t.T
